# Initial kernel scaffold; baseline (speedup 1.0000x reference)
import jax
import jax.numpy as jnp
from jax.experimental import pallas as pl


def kernel(x, expert_weights_v, expert_weights_g, gate_w, gate_b):
    raise NotImplementedError("write your pallas kernel here")


# speedup vs baseline: 2.9600x; 2.9600x over previous
import functools

import jax
import jax.numpy as jnp
from jax.experimental import pallas as pl

B, D, H, E, TOP_K = 2048, 1024, 2048, 8, 2
TB = 512
NB = B // TB


def _moe_body(x_ref, wv_ref, wg_ref, gw_ref, gb_ref, o_ref):
    e = pl.program_id(1)
    x = x_ref[...]
    logits = jnp.dot(x, gw_ref[...].T, preferred_element_type=jnp.float32)
    logits = logits + gb_ref[...]
    m = jnp.max(logits, axis=1, keepdims=True)
    ex = jnp.exp(logits - m)
    probs = ex / jnp.sum(ex, axis=1, keepdims=True)
    col = jax.lax.broadcasted_iota(jnp.int32, probs.shape, 1)
    p_e = jnp.sum(jnp.where(col == e, probs, 0.0), axis=1, keepdims=True)
    gt = (probs > p_e).astype(jnp.float32)
    eq_lt = ((probs == p_e) & (col < e)).astype(jnp.float32)
    rank = jnp.sum(gt + eq_lt, axis=1)
    w = jnp.where(rank < TOP_K, p_e[:, 0], 0.0)

    v = jnp.dot(x, wv_ref[0], preferred_element_type=jnp.float32)
    g = jnp.dot(x, wg_ref[0], preferred_element_type=jnp.float32)
    contrib = (v * jax.nn.sigmoid(g)) * w[:, None]

    @pl.when(e == 0)
    def _():
        o_ref[...] = contrib

    @pl.when(e != 0)
    def _():
        o_ref[...] += contrib


@functools.partial(jax.jit, static_argnames=())
def kernel(x, expert_weights_v, expert_weights_g, gate_w, gate_b):
    gb2 = gate_b.reshape(1, E)
    return pl.pallas_call(
        _moe_body,
        grid=(NB, E),
        in_specs=[
            pl.BlockSpec((TB, D), lambda b, e: (b, 0)),
            pl.BlockSpec((1, D, H), lambda b, e: (e, 0, 0)),
            pl.BlockSpec((1, D, H), lambda b, e: (e, 0, 0)),
            pl.BlockSpec((E, D), lambda b, e: (0, 0)),
            pl.BlockSpec((1, E), lambda b, e: (0, 0)),
        ],
        out_specs=pl.BlockSpec((TB, H), lambda b, e: (b, 0)),
        out_shape=jax.ShapeDtypeStruct((B, H), jnp.float32),
    )(x, expert_weights_v, expert_weights_g, gate_w, gb2)
